# baseline (speedup 1.0000x reference)
import functools

import jax
import jax.numpy as jnp
from jax.experimental import pallas as pl
from jax.experimental.pallas import tpu as pltpu

_K0 = 908003072
_K1 = 3252900185

_ROWS = 128
_VOCAB = 100000
_BLOCK_COLS = 2048


def _threefry2x32_zero_hi(x1):
    ks0 = jnp.uint32(_K0)
    ks1 = jnp.uint32(_K1)
    ks2 = jnp.uint32(_K0 ^ _K1 ^ 0x1BD11BDA)
    ks = (ks0, ks1, ks2)
    rot = ((13, 15, 26, 6), (17, 29, 16, 24))
    x0 = jnp.full_like(x1, ks0)
    x1 = x1 + ks1
    for i in range(5):
        for r in rot[i % 2]:
            x0 = x0 + x1
            x1 = (x1 << r) | (x1 >> (32 - r))
            x1 = x0 ^ x1
        x0 = x0 + ks[(i + 1) % 3]
        x1 = x1 + ks[(i + 2) % 3] + jnp.uint32(i + 1)
    return x0 ^ x1


def _first_argmax(vals, col_idx):
    vmax = jnp.max(vals, axis=1, keepdims=True)
    idxs = jnp.where(vals == vmax, col_idx, jnp.int32(2**31 - 1))
    vidx = jnp.min(idxs, axis=1, keepdims=True)
    return vmax, vidx


def _sampler_kernel(nblocks, lg_ref, t_ref, out_ref,
                    gval, gidx, sval, sidx):
    i = pl.program_id(0)
    lg = lg_ref[...]
    bc = lg.shape[1]
    col_idx = jax.lax.broadcasted_iota(jnp.int32, lg.shape, 1) + i * bc
    valid = col_idx < _VOCAB
    lg = jnp.where(valid, lg, -jnp.inf)

    bgv, bgi = _first_argmax(lg, col_idx)

    row_idx = jax.lax.broadcasted_iota(jnp.int32, lg.shape, 0)
    cnt = (row_idx * _VOCAB + col_idx).astype(jnp.uint32)
    bits = _threefry2x32_zero_hi(cnt)
    u = jax.lax.bitcast_convert_type(
        (bits >> 9) | jnp.uint32(0x3F800000), jnp.float32) - 1.0
    expo = jnp.maximum(-jnp.log1p(-u), jnp.float32(1e-10))

    t = t_ref[...]
    safe_t = jnp.where(t <= 1e-10, jnp.ones_like(t), t)
    score = lg / safe_t - jnp.log(expo)
    score = jnp.where(valid, score, -jnp.inf)
    bsv, bsi = _first_argmax(score, col_idx)

    @pl.when(i == 0)
    def _init():
        gval[...] = jnp.full_like(gval, -jnp.inf)
        gidx[...] = jnp.zeros_like(gidx)
        sval[...] = jnp.full_like(sval, -jnp.inf)
        sidx[...] = jnp.zeros_like(sidx)

    gb = bgv > gval[...]
    gval[...] = jnp.where(gb, bgv, gval[...])
    gidx[...] = jnp.where(gb, bgi, gidx[...])
    sb = bsv > sval[...]
    sval[...] = jnp.where(sb, bsv, sval[...])
    sidx[...] = jnp.where(sb, bsi, sidx[...])

    @pl.when(i == nblocks - 1)
    def _finish():
        greedy = t <= 1e-10
        out_ref[...] = jnp.where(greedy, gidx[...], sidx[...])


@jax.jit
def kernel(logits, temperatures):
    lg = logits.astype(jnp.float32)
    t2 = temperatures.reshape(_ROWS, 1)
    nblocks = pl.cdiv(_VOCAB, _BLOCK_COLS)
    out = pl.pallas_call(
        functools.partial(_sampler_kernel, nblocks),
        grid=(nblocks,),
        in_specs=[
            pl.BlockSpec((_ROWS, _BLOCK_COLS), lambda i: (0, i)),
            pl.BlockSpec((_ROWS, 1), lambda i: (0, 0)),
        ],
        out_specs=pl.BlockSpec((_ROWS, 1), lambda i: (0, 0)),
        out_shape=jax.ShapeDtypeStruct((_ROWS, 1), jnp.int32),
        scratch_shapes=[
            pltpu.VMEM((_ROWS, 1), jnp.float32),
            pltpu.VMEM((_ROWS, 1), jnp.int32),
            pltpu.VMEM((_ROWS, 1), jnp.float32),
            pltpu.VMEM((_ROWS, 1), jnp.int32),
        ],
    )(lg, t2)
    return out.reshape(_ROWS)


# speedup vs baseline: 1.0329x; 1.0329x over previous
import functools

import jax
import jax.numpy as jnp
from jax.experimental import pallas as pl
from jax.experimental.pallas import tpu as pltpu

_K0 = 908003072
_K1 = 3252900185

_ROWS = 128
_VOCAB = 100000
_BLOCK_COLS = 2048


def _threefry2x32_zero_hi(x1):
    ks0 = jnp.uint32(_K0)
    ks1 = jnp.uint32(_K1)
    ks2 = jnp.uint32(_K0 ^ _K1 ^ 0x1BD11BDA)
    ks = (ks0, ks1, ks2)
    rot = ((13, 15, 26, 6), (17, 29, 16, 24))
    x0 = jnp.full_like(x1, ks0)
    x1 = x1 + ks1
    for i in range(5):
        for r in rot[i % 2]:
            x0 = x0 + x1
            x1 = (x1 << r) | (x1 >> (32 - r))
            x1 = x0 ^ x1
        x0 = x0 + ks[(i + 1) % 3]
        x1 = x1 + ks[(i + 2) % 3] + jnp.uint32(i + 1)
    return x0 ^ x1


def _first_argmax(vals, col_idx):
    vmax = jnp.max(vals, axis=1, keepdims=True)
    idxs = jnp.where(vals == vmax, col_idx, jnp.int32(2**31 - 1))
    vidx = jnp.min(idxs, axis=1, keepdims=True)
    return vmax, vidx


def _sampler_kernel(nblocks, lg_ref, t_ref, out_ref, sval, sidx):
    i = pl.program_id(0)
    lg = lg_ref[...]
    bc = lg.shape[1]
    col_idx = jax.lax.broadcasted_iota(jnp.int32, lg.shape, 1) + i * bc
    lg = jnp.where(col_idx < _VOCAB, lg, -jnp.inf)

    row_idx = jax.lax.broadcasted_iota(jnp.int32, lg.shape, 0)
    cnt = (row_idx * _VOCAB + col_idx).astype(jnp.uint32)
    bits = _threefry2x32_zero_hi(cnt)
    u = jax.lax.bitcast_convert_type(
        (bits >> 9) | jnp.uint32(0x3F800000), jnp.float32) - 1.0
    expo = jnp.maximum(-jnp.log1p(-u), jnp.float32(1e-10))

    t = t_ref[...]
    greedy = t <= 1e-10
    rcp = 1.0 / jnp.where(greedy, jnp.ones_like(t), t)
    score = jnp.where(greedy, lg, lg * rcp - jnp.log(expo))
    bsv, bsi = _first_argmax(score, col_idx)

    @pl.when(i == 0)
    def _init():
        sval[...] = jnp.full_like(sval, -jnp.inf)
        sidx[...] = jnp.zeros_like(sidx)

    sb = bsv > sval[...]
    sval[...] = jnp.where(sb, bsv, sval[...])
    sidx[...] = jnp.where(sb, bsi, sidx[...])

    @pl.when(i == nblocks - 1)
    def _finish():
        out_ref[...] = sidx[...]


@jax.jit
def kernel(logits, temperatures):
    lg = logits.astype(jnp.float32)
    t2 = temperatures.reshape(_ROWS, 1)
    nblocks = pl.cdiv(_VOCAB, _BLOCK_COLS)
    out = pl.pallas_call(
        functools.partial(_sampler_kernel, nblocks),
        grid=(nblocks,),
        in_specs=[
            pl.BlockSpec((_ROWS, _BLOCK_COLS), lambda i: (0, i)),
            pl.BlockSpec((_ROWS, 1), lambda i: (0, 0)),
        ],
        out_specs=pl.BlockSpec((_ROWS, 1), lambda i: (0, 0)),
        out_shape=jax.ShapeDtypeStruct((_ROWS, 1), jnp.int32),
        scratch_shapes=[
            pltpu.VMEM((_ROWS, 1), jnp.float32),
            pltpu.VMEM((_ROWS, 1), jnp.int32),
        ],
    )(lg, t2)
    return out.reshape(_ROWS)
